# baseline (speedup 1.0000x reference)
import functools

import jax
import jax.numpy as jnp
from jax import lax
from jax.experimental import pallas as pl
from jax.experimental.pallas import tpu as pltpu
from jax.experimental.pallas import tpu_sc as plsc

NUM_CLASSES = 1000000
EMBED_DIM = 64
BATCH = 16384

NC = 2
NS = 16
NW = NC * NS
B_PER_W = BATCH // NW
CHUNK = 64
NCH = B_PER_W // CHUNK
GRP = 8


def _sc_gather_body(table_hbm, idx_hbm, out_hbm, idx_v, rows_v, sem_g, sem_o):
    wid = lax.axis_index("s") * NC + lax.axis_index("c")
    pltpu.sync_copy(idx_hbm.at[wid], idx_v)

    oh = [None] * NCH
    for j in range(NCH):
        b = j % 2
        gh = []
        if j >= 2:
            oh[j - 2].wait()
        for k in range(CHUNK // 16):
            vec = idx_v[pl.ds(j * CHUNK + k * 16, 16)]
            for l in range(16):
                gh.append(
                    pltpu.async_copy(
                        table_hbm.at[vec[l]],
                        rows_v.at[b, k * 16 + l], sem_g))
        for h in gh:
            h.wait()
        oh[j] = pltpu.async_copy(rows_v.at[b], out_hbm.at[wid, j], sem_o)
    oh[NCH - 2].wait()
    oh[NCH - 1].wait()


@jax.jit
def _sc_gather(table_hbm, idx2d):
    mesh = plsc.VectorSubcoreMesh(core_axis_name="c", subcore_axis_name="s")
    fn = pl.kernel(
        _sc_gather_body,
        out_type=jax.ShapeDtypeStruct((NW, NCH, CHUNK, EMBED_DIM),
                                      jnp.float32),
        mesh=mesh,
        scratch_types=[
            pltpu.VMEM((B_PER_W,), jnp.int32),
            pltpu.VMEM((2, CHUNK, EMBED_DIM), jnp.float32),
            pltpu.SemaphoreType.DMA,
            pltpu.SemaphoreType.DMA,
        ],
    )
    return fn(table_hbm, idx2d)


def _tc_linear_body(x_ref, wt_ref, b_ref, o_ref):
    x = x_ref[...]
    s = x * jax.nn.sigmoid(x)
    o_ref[...] = (
        jnp.dot(s, wt_ref[...], preferred_element_type=jnp.float32) + b_ref[...]
    )


@jax.jit
def _tc_linear(x, wt, b2d):
    blk = 2048
    grid = (BATCH // blk,)
    return pl.pallas_call(
        _tc_linear_body,
        grid=grid,
        in_specs=[
            pl.BlockSpec((blk, EMBED_DIM), lambda i: (i, 0)),
            pl.BlockSpec((EMBED_DIM, EMBED_DIM), lambda i: (0, 0)),
            pl.BlockSpec((1, EMBED_DIM), lambda i: (0, 0)),
        ],
        out_specs=pl.BlockSpec((blk, EMBED_DIM), lambda i: (i, 0)),
        out_shape=jax.ShapeDtypeStruct((BATCH, EMBED_DIM), jnp.float32),
    )(x, wt, b2d)


def kernel(class_labels, table, W, b):
    idx2d = class_labels.astype(jnp.int32).reshape(NW, B_PER_W)
    gathered = _sc_gather(table, idx2d)
    x = gathered.reshape(BATCH, EMBED_DIM)
    return _tc_linear(x, W.T, b.reshape(1, EMBED_DIM))


# speedup vs baseline: 1.5185x; 1.5185x over previous
import functools

import jax
import jax.numpy as jnp
from jax import lax
from jax.experimental import pallas as pl
from jax.experimental.pallas import tpu as pltpu
from jax.experimental.pallas import tpu_sc as plsc

NUM_CLASSES = 1000000
EMBED_DIM = 64
BATCH = 16384

NC = 2
NS = 16
NW = NC * NS
B_PER_W = BATCH // NW
SB = 4
SB_LAB = B_PER_W // SB
NG = SB_LAB // 16


def _sc_gather_body(tableT, idx_hbm, outT_hbm, idx_v, half_v, outT_v,
                    sem_g, sem_o):
    wid = lax.axis_index("s") * NC + lax.axis_index("c")
    base = wid * B_PER_W
    pltpu.sync_copy(idx_hbm.at[wid], idx_v)
    lane = lax.iota(jnp.int32, 16)

    def superblock(sb):
        for g in range(NG):
            vec = idx_v[pl.ds(sb * SB_LAB + g * 16, 16)]
            tbv = jnp.right_shift(vec, 7)
            phv = jnp.bitwise_and(vec, 127)
            for half in range(2):
                gh = []
                for l in range(16):
                    off = pl.multiple_of(tbv[l] * 128, 128)
                    gh.append(pltpu.async_copy(
                        tableT.at[pl.ds(half * 32, 32), pl.ds(off, 128)],
                        half_v.at[l], sem_g))
                for h in gh:
                    h.wait()
                for c in range(32):
                    vals = plsc.load_gather(
                        half_v, [lane, jnp.full((16,), c, jnp.int32), phv])
                    outT_v[half * 32 + c, pl.ds(g * 16, 16)] = vals
        col0 = pl.multiple_of(base + sb * SB_LAB, 128)
        pltpu.sync_copy(outT_v, outT_hbm.at[:, pl.ds(col0, SB_LAB)])

    pl.loop(0, SB)(superblock)


@jax.jit
def _sc_gather(tableT, idx2d):
    mesh = plsc.VectorSubcoreMesh(core_axis_name="c", subcore_axis_name="s")
    fn = pl.kernel(
        _sc_gather_body,
        out_type=jax.ShapeDtypeStruct((EMBED_DIM, BATCH), jnp.float32),
        mesh=mesh,
        scratch_types=[
            pltpu.VMEM((B_PER_W,), jnp.int32),
            pltpu.VMEM((16, 32, 128), jnp.float32),
            pltpu.VMEM((EMBED_DIM, SB_LAB), jnp.float32),
            pltpu.SemaphoreType.DMA,
            pltpu.SemaphoreType.DMA,
        ],
        compiler_params=pltpu.CompilerParams(needs_layout_passes=False),
    )
    return fn(tableT, idx2d)


def _tc_linear_body(xt_ref, w_ref, b_ref, o_ref):
    x = xt_ref[...]
    s = x * jax.nn.sigmoid(x)
    o_ref[...] = (
        jax.lax.dot_general(w_ref[...], s, (((1,), (0,)), ((), ())),
                            preferred_element_type=jnp.float32) + b_ref[...]
    )


@jax.jit
def _tc_linear_t(xt, W, bcol):
    blk = 4096
    grid = (BATCH // blk,)
    return pl.pallas_call(
        _tc_linear_body,
        grid=grid,
        in_specs=[
            pl.BlockSpec((EMBED_DIM, blk), lambda i: (0, i)),
            pl.BlockSpec((EMBED_DIM, EMBED_DIM), lambda i: (0, 0)),
            pl.BlockSpec((EMBED_DIM, 1), lambda i: (0, 0)),
        ],
        out_specs=pl.BlockSpec((EMBED_DIM, blk), lambda i: (0, i)),
        out_shape=jax.ShapeDtypeStruct((EMBED_DIM, BATCH), jnp.float32),
    )(xt, W, bcol)


def kernel(class_labels, table, W, b):
    idx2d = class_labels.astype(jnp.int32).reshape(NW, B_PER_W)
    xt = _sc_gather(table.T, idx2d)
    ot = _tc_linear_t(xt, W, b.reshape(EMBED_DIM, 1))
    return ot.T
